# Initial kernel scaffold; baseline (speedup 1.0000x reference)
import jax
import jax.numpy as jnp
from jax import lax
from jax.experimental import pallas as pl
from jax.experimental.pallas import tpu as pltpu
from jax.experimental.pallas import tpu_sc as plsc

SEQ_LEN = 200
EMBED_DIM = 64
BATCH = 4096
NUM_ROWS = BATCH * SEQ_LEN

NC, NS, LANES = 2, 16, 16
NW = NC * NS
ROWS_PER_W = NUM_ROWS // NW
CHUNK = SEQ_LEN
NCHUNKS = ROWS_PER_W // CHUNK


def _body(idx_hbm, tok_hbm, pos_hbm, out_hbm, idx_v, rows_v, pos_v, sem):
    wid = lax.axis_index("s") * NC + lax.axis_index("c")
    base = wid * ROWS_PER_W
    pltpu.sync_copy(pos_hbm, pos_v)

    def chunk_body(k, carry):
        start = base + k * CHUNK
        pltpu.sync_copy(idx_hbm.at[pl.ds(start, CHUNK)], idx_v)
        pltpu.async_copy(tok_hbm.at[idx_v], rows_v, sem).wait()

        def row_body(r, c):
            for j in range(EMBED_DIM // LANES):
                s = pl.ds(j * LANES, LANES)
                rows_v[r, s] = rows_v[r, s] + pos_v[r, s]
            return c

        lax.fori_loop(0, CHUNK, row_body, 0)
        pltpu.sync_copy(rows_v, out_hbm.at[pl.ds(start, CHUNK)])
        return carry

    lax.fori_loop(0, NCHUNKS, chunk_body, 0)


_mesh = plsc.VectorSubcoreMesh(core_axis_name="c", subcore_axis_name="s")

_gather = pl.kernel(
    _body,
    out_type=jax.ShapeDtypeStruct((NUM_ROWS, EMBED_DIM), jnp.float32),
    mesh=_mesh,
    scratch_types=[
        pltpu.VMEM((CHUNK,), jnp.int32),
        pltpu.VMEM((CHUNK, EMBED_DIM), jnp.float32),
        pltpu.VMEM((SEQ_LEN, EMBED_DIM), jnp.float32),
        pltpu.SemaphoreType.DMA,
    ],
)


@jax.jit
def kernel(inputs, token_table, position_table):
    idx = inputs.reshape(-1).astype(jnp.int32)
    out = _gather(idx, token_table, position_table)
    return out.reshape(BATCH, SEQ_LEN, EMBED_DIM)


# speedup vs baseline: 3.1000x; 3.1000x over previous
import jax
import jax.numpy as jnp
from jax import lax
from jax.experimental import pallas as pl
from jax.experimental.pallas import tpu as pltpu
from jax.experimental.pallas import tpu_sc as plsc

SEQ_LEN = 200
EMBED_DIM = 64
BATCH = 4096
NUM_ROWS = BATCH * SEQ_LEN

NC, NS, LANES = 2, 16, 16
NW = NC * NS
ROWS_PER_W = NUM_ROWS // NW
CHUNK = SEQ_LEN
NCHUNKS = ROWS_PER_W // CHUNK


def _body(idx_hbm, tok_hbm, pos_hbm, out_hbm, idx_v, rows_v, pos_v, sem):
    wid = lax.axis_index("s") * NC + lax.axis_index("c")
    base = wid * ROWS_PER_W
    pltpu.sync_copy(pos_hbm, pos_v)

    def chunk_body(k, carry):
        start = base + k * CHUNK
        pltpu.sync_copy(idx_hbm.at[pl.ds(start, CHUNK)], idx_v)
        pltpu.async_copy(tok_hbm.at[idx_v], rows_v, sem).wait()

        def row_body(r, c):
            for j in range(EMBED_DIM // LANES):
                s = pl.ds(j * LANES, LANES)
                rows_v[r, s] = rows_v[r, s] + pos_v[r, s]
            return c

        lax.fori_loop(0, CHUNK, row_body, 0)
        pltpu.sync_copy(rows_v, out_hbm.at[pl.ds(start, CHUNK)])
        return carry

    lax.fori_loop(0, NCHUNKS, chunk_body, 0)


_mesh = plsc.VectorSubcoreMesh(core_axis_name="c", subcore_axis_name="s")

_gather = pl.kernel(
    _body,
    out_type=jax.ShapeDtypeStruct((NUM_ROWS, EMBED_DIM), jnp.float32),
    mesh=_mesh,
    scratch_types=[
        pltpu.VMEM((CHUNK,), jnp.int32),
        pltpu.VMEM((CHUNK, EMBED_DIM), jnp.float32),
        pltpu.VMEM((SEQ_LEN, EMBED_DIM), jnp.float32),
        pltpu.SemaphoreType.DMA,
    ],
    compiler_params=pltpu.CompilerParams(use_tc_tiling_on_sc=False),
)


@jax.jit
def kernel(inputs, token_table, position_table):
    idx = inputs.reshape(-1).astype(jnp.int32)
    out = _gather(idx, token_table, position_table)
    return out.reshape(BATCH, SEQ_LEN, EMBED_DIM)


# speedup vs baseline: 3.9340x; 1.2690x over previous
import jax
import jax.numpy as jnp
from jax import lax
from jax.experimental import pallas as pl
from jax.experimental.pallas import tpu as pltpu
from jax.experimental.pallas import tpu_sc as plsc

SEQ_LEN = 200
EMBED_DIM = 64
BATCH = 4096
NUM_ROWS = BATCH * SEQ_LEN

NC, NS = 2, 16
NW = NC * NS
ROWS_PER_W = NUM_ROWS // NW
CHUNK = SEQ_LEN
NCHUNKS = ROWS_PER_W // CHUNK
NBUF = 4


def _body(idx_hbm, tok_hbm, pos_hbm, out_hbm, idx_v, rows_v, pos_v, g_sem, o_sem):
    wid = lax.axis_index("s") * NC + lax.axis_index("c")
    base = wid * ROWS_PER_W
    pltpu.sync_copy(pos_hbm, pos_v)

    def start_gather(k, b):
        start = base + k * CHUNK
        pltpu.sync_copy(idx_hbm.at[pl.ds(start, CHUNK)], idx_v.at[b])
        pltpu.async_copy(tok_hbm.at[idx_v.at[b]], rows_v.at[b], g_sem.at[b])

    def wait_gather(b):
        pltpu.make_async_copy(tok_hbm.at[idx_v.at[b]], rows_v.at[b],
                              g_sem.at[b]).wait()

    def start_out(k, b):
        start = base + k * CHUNK
        pltpu.async_copy(rows_v.at[b], out_hbm.at[pl.ds(start, CHUNK)],
                         o_sem.at[b])

    def wait_out(k, b):
        start = base + k * CHUNK
        pltpu.make_async_copy(rows_v.at[b], out_hbm.at[pl.ds(start, CHUNK)],
                              o_sem.at[b]).wait()

    start_gather(0, 0)
    start_gather(1, 1)

    def step(i, b):
        k = NBUF * i + b
        b2 = (b + 2) % NBUF

        def prefetch():
            def drain_prev_out():
                wait_out(k - 2, b2)

            if b >= 2:
                drain_prev_out()
            else:
                lax.cond(i >= 1, drain_prev_out, lambda: None)
            start_gather(k + 2, b2)

        if b < 2:
            prefetch()
        else:
            lax.cond(i < NCHUNKS // NBUF - 1, prefetch, lambda: None)

        wait_gather(b)

        def add_pos(r, c):
            for j in range(EMBED_DIM // 16):
                s = pl.ds(j * 16, 16)
                plsc.addupdate(rows_v.at[b, r, s], pos_v[r, s])
            return c

        lax.fori_loop(0, CHUNK, add_pos, 0)
        start_out(k, b)

    def outer(i, carry):
        for b in range(NBUF):
            step(i, b)
        return carry

    lax.fori_loop(0, NCHUNKS // NBUF, outer, 0)

    for b in range(NBUF):
        wait_out(NCHUNKS - NBUF + b, b)


_mesh = plsc.VectorSubcoreMesh(core_axis_name="c", subcore_axis_name="s")

_gather = pl.kernel(
    _body,
    out_type=jax.ShapeDtypeStruct((NUM_ROWS, EMBED_DIM), jnp.float32),
    mesh=_mesh,
    scratch_types=[
        pltpu.VMEM((NBUF, CHUNK), jnp.int32),
        pltpu.VMEM((NBUF, CHUNK, EMBED_DIM), jnp.float32),
        pltpu.VMEM((SEQ_LEN, EMBED_DIM), jnp.float32),
        pltpu.SemaphoreType.DMA((NBUF,)),
        pltpu.SemaphoreType.DMA((NBUF,)),
    ],
    compiler_params=pltpu.CompilerParams(use_tc_tiling_on_sc=False),
)


@jax.jit
def kernel(inputs, token_table, position_table):
    idx = inputs.reshape(-1).astype(jnp.int32)
    out = _gather(idx, token_table, position_table)
    return out.reshape(BATCH, SEQ_LEN, EMBED_DIM)
